# Initial kernel scaffold; baseline (speedup 1.0000x reference)
import jax
import jax.numpy as jnp
from jax.experimental import pallas as pl


def kernel(x):
    raise NotImplementedError("write your pallas kernel here")


# speedup vs baseline: 10.7726x; 10.7726x over previous
import jax
import jax.numpy as jnp
from jax import lax
from jax.experimental import pallas as pl
from jax.experimental.pallas import tpu as pltpu
from jax.experimental.pallas import tpu_sc as plsc

K = 64
N_ROWS = 128
N_COLS = 32768
NC = 2
NS = 16
NW = NC * NS
ROWS_PER_W = N_ROWS // NW
CHUNKS = N_COLS // K


def _sortd(v):
    s, _ = plsc.sort_key_val(v, v, descending=True)
    return s


def _rev(v):
    return lax.rev(v, (0,))


def _merge2(a, b):
    br = _rev(b)
    hi = jnp.maximum(a, br)
    lo = jnp.minimum(a, br)
    return _sortd(hi), _sortd(lo)


def _merge32(a0, a1, b0, b1):
    rb0, rb1 = _rev(b1), _rev(b0)
    hi0 = jnp.maximum(a0, rb0)
    hi1 = jnp.maximum(a1, rb1)
    lo0 = jnp.minimum(a0, rb0)
    lo1 = jnp.minimum(a1, rb1)
    h0 = jnp.maximum(hi0, hi1)
    h1 = jnp.minimum(hi0, hi1)
    l0 = jnp.maximum(lo0, lo1)
    l1 = jnp.minimum(lo0, lo1)
    return _sortd(h0), _sortd(h1), _sortd(l0), _sortd(l1)


def _sort64(c0, c1, c2, c3):
    a0, a1 = _merge2(_sortd(c0), _sortd(c1))
    b0, b1 = _merge2(_sortd(c2), _sortd(c3))
    return _merge32(a0, a1, b0, b1)


def _merge_top64(t, c):
    t0, t1, t2, t3 = t
    c0, c1, c2, c3 = c
    h0 = jnp.maximum(t0, _rev(c3))
    h1 = jnp.maximum(t1, _rev(c2))
    h2 = jnp.maximum(t2, _rev(c1))
    h3 = jnp.maximum(t3, _rev(c0))
    p0 = jnp.maximum(h0, h2)
    p2 = jnp.minimum(h0, h2)
    p1 = jnp.maximum(h1, h3)
    p3 = jnp.minimum(h1, h3)
    q0 = jnp.maximum(p0, p1)
    q1 = jnp.minimum(p0, p1)
    q2 = jnp.maximum(p2, p3)
    q3 = jnp.minimum(p2, p3)
    return _sortd(q0), _sortd(q1), _sortd(q2), _sortd(q3)


def _tec_body(x_hbm, out_hbm, row_v, out_v):
    wid = lax.axis_index("s") * NC + lax.axis_index("c")
    row0 = wid * ROWS_PER_W
    for i in range(ROWS_PER_W):
        pltpu.sync_copy(x_hbm.at[row0 + i], row_v)

        def chunk_body(j, t):
            base = j * K
            c0 = row_v[pl.ds(base, 16)]
            c1 = row_v[pl.ds(base + 16, 16)]
            c2 = row_v[pl.ds(base + 32, 16)]
            c3 = row_v[pl.ds(base + 48, 16)]
            return _merge_top64(t, _sort64(c0, c1, c2, c3))

        neg = jnp.full((16,), -jnp.inf, jnp.float32)
        t = lax.fori_loop(0, CHUNKS, chunk_body, (neg, neg, neg, neg))
        for k in range(4):
            out_v[i, pl.ds(16 * k, 16)] = t[k]
    pltpu.sync_copy(out_v, out_hbm.at[pl.ds(row0, ROWS_PER_W)])


def kernel(x):
    mesh = plsc.VectorSubcoreMesh(core_axis_name="c", subcore_axis_name="s")
    run = pl.kernel(
        _tec_body,
        mesh=mesh,
        out_type=jax.ShapeDtypeStruct((N_ROWS, K), jnp.float32),
        scratch_types=[
            pltpu.VMEM((N_COLS,), jnp.float32),
            pltpu.VMEM((ROWS_PER_W, K), jnp.float32),
        ],
        compiler_params=pltpu.CompilerParams(needs_layout_passes=False),
    )
    return run(x)
